# baseline (speedup 1.0000x reference)
import functools
import math

import jax
import jax.numpy as jnp
from jax import lax
from jax.experimental import pallas as pl
from jax.experimental.pallas import tpu as pltpu
from jax.experimental.pallas import tpu_sc as plsc

D_MODEL = 64
SCALE = math.sqrt(D_MODEL)

_info = plsc.get_sparse_core_info()
_NC, _NS, _L = _info.num_cores, _info.num_subcores, _info.num_lanes
_NW = _NC * _NS

IDX_W = 128
CHUNK = 4
ROWS_PER_CHUNK = CHUNK * IDX_W


def _sc_embed(table, idx2d):
  n_idx_rows = idx2d.shape[0]
  rows_per_worker = n_idx_rows // _NW
  chunks = rows_per_worker // CHUNK
  total_rows = n_idx_rows * IDX_W

  mesh = plsc.VectorSubcoreMesh(core_axis_name="c", subcore_axis_name="s")

  @functools.partial(
      pl.kernel,
      mesh=mesh,
      out_type=jax.ShapeDtypeStruct((total_rows, D_MODEL), jnp.float32),
      scratch_types=[
          pltpu.VMEM((CHUNK, IDX_W), jnp.int32),
          pltpu.VMEM((ROWS_PER_CHUNK, D_MODEL), jnp.float32),
          pltpu.SemaphoreType.DMA,
      ],
      compiler_params=pltpu.CompilerParams(use_tc_tiling_on_sc=False),
  )
  def k(table_hbm, idx_hbm, out_hbm, idx_v, rows_v, sem):
    wid = lax.axis_index("s") * _NC + lax.axis_index("c")
    row_base = wid * rows_per_worker

    def chunk_body(ci, carry):
      base = row_base + ci * CHUNK
      pltpu.sync_copy(idx_hbm.at[pl.ds(base, CHUNK)], idx_v)
      copies = []
      for j in range(CHUNK):
        copies.append(
            pltpu.async_copy(
                table_hbm.at[idx_v.at[j]],
                rows_v.at[pl.ds(j * IDX_W, IDX_W)],
                sem,
            ))
      for c in copies:
        c.wait()

      def scale_body(r, c2):
        for j in range(D_MODEL // _L):
          v = rows_v[r, pl.ds(j * _L, _L)]
          rows_v[r, pl.ds(j * _L, _L)] = v * SCALE
        return c2

      lax.fori_loop(0, ROWS_PER_CHUNK, scale_body, 0, unroll=4)
      pltpu.sync_copy(rows_v, out_hbm.at[pl.ds(base * IDX_W, ROWS_PER_CHUNK)])
      return carry

    lax.fori_loop(0, chunks, chunk_body, 0)

  return k(table, idx2d)


def kernel(x, table):
  b, s = x.shape
  idx2d = x.reshape(-1, IDX_W).astype(jnp.int32)
  out = _sc_embed(table, idx2d)
  return out.reshape(b, s, D_MODEL)


# speedup vs baseline: 1.0935x; 1.0935x over previous
import functools
import math

import jax
import jax.numpy as jnp
from jax import lax
from jax.experimental import pallas as pl
from jax.experimental.pallas import tpu as pltpu
from jax.experimental.pallas import tpu_sc as plsc

D_MODEL = 64
SCALE = math.sqrt(D_MODEL)

_info = plsc.get_sparse_core_info()
_NC, _NS, _L = _info.num_cores, _info.num_subcores, _info.num_lanes
_NW = _NC * _NS

IDX_W = 128
CHUNK = 2
ROWS_PER_CHUNK = CHUNK * IDX_W
NBUF = 4
LOOKAHEAD = 2


def _sc_embed(table, idx2d):
  n_idx_rows = idx2d.shape[0]
  rows_per_worker = n_idx_rows // _NW
  chunks = rows_per_worker // CHUNK
  assert chunks % NBUF == 0
  total_rows = n_idx_rows * IDX_W

  mesh = plsc.VectorSubcoreMesh(core_axis_name="c", subcore_axis_name="s")

  @functools.partial(
      pl.kernel,
      mesh=mesh,
      out_type=jax.ShapeDtypeStruct((total_rows, D_MODEL), jnp.float32),
      scratch_types=[
          pltpu.VMEM((rows_per_worker, IDX_W), jnp.int32),
      ] + [pltpu.VMEM((ROWS_PER_CHUNK, D_MODEL), jnp.float32)] * NBUF
        + [pltpu.SemaphoreType.DMA] * (2 * NBUF),
      compiler_params=pltpu.CompilerParams(use_tc_tiling_on_sc=False),
  )
  def k(table_hbm, idx_hbm, out_hbm, idx_all, *bufs_and_sems):
    rows_v = bufs_and_sems[:NBUF]
    gsem = bufs_and_sems[NBUF:2 * NBUF]
    ssem = bufs_and_sems[2 * NBUF:]

    wid = lax.axis_index("s") * _NC + lax.axis_index("c")
    out_base0 = wid * rows_per_worker * IDX_W

    pltpu.sync_copy(idx_hbm.at[pl.ds(wid * rows_per_worker, rows_per_worker)],
                    idx_all)

    def fire_gathers(m, b):
      for j in range(CHUNK):
        pltpu.async_copy(
            table_hbm.at[idx_all.at[m * CHUNK + j]],
            rows_v[b].at[pl.ds(j * IDX_W, IDX_W)],
            gsem[b],
        )

    def drain_store(b):
      pltpu.make_async_copy(
          rows_v[b], out_hbm.at[pl.ds(0, ROWS_PER_CHUNK)], ssem[b]).wait()

    def process(ci, b):
      for j in range(CHUNK):
        pltpu.make_async_copy(
            table_hbm.at[idx_all.at[ci * CHUNK + j]],
            rows_v[b].at[pl.ds(j * IDX_W, IDX_W)],
            gsem[b],
        ).wait()

      def scale_body(r, c2):
        for j in range(D_MODEL // _L):
          v = rows_v[b][r, pl.ds(j * _L, _L)]
          rows_v[b][r, pl.ds(j * _L, _L)] = v * SCALE
        return c2

      lax.fori_loop(0, ROWS_PER_CHUNK, scale_body, 0, unroll=8)
      pltpu.async_copy(
          rows_v[b],
          out_hbm.at[pl.ds(out_base0 + ci * ROWS_PER_CHUNK, ROWS_PER_CHUNK)],
          ssem[b],
      )

    for m in range(LOOKAHEAD):
      fire_gathers(m, m % NBUF)

    def group_body(g, carry):
      for b in range(NBUF):
        ci = g * NBUF + b
        m = ci + LOOKAHEAD
        bm = (b + LOOKAHEAD) % NBUF

        @pl.when(m < chunks)
        def _():
          @pl.when(m >= NBUF)
          def _():
            drain_store(bm)
          fire_gathers(m, bm)

        process(ci, b)
      return carry

    lax.fori_loop(0, chunks // NBUF, group_body, 0)

    for b in range(NBUF):
      drain_store(b)

  return k(table, idx2d)


def kernel(x, table):
  b, s = x.shape
  idx2d = x.reshape(-1, IDX_W).astype(jnp.int32)
  out = _sc_embed(table, idx2d)
  return out.reshape(b, s, D_MODEL)
